# Initial kernel scaffold; baseline (speedup 1.0000x reference)
import jax
import jax.numpy as jnp
from jax.experimental import pallas as pl


def kernel(batch, feat, W0, b0, Ws, bs, W1, b1):
    raise NotImplementedError("write your pallas kernel here")


# speedup vs baseline: 3.2852x; 3.2852x over previous
import functools

import jax
import jax.numpy as jnp
from jax.experimental import pallas as pl

_NUM_LAYERS = 4
_BLOCK_ROWS = 2000


def _mlp_chain_kernel(x_ref, w0_ref, b0_ref, ws_ref, bs_ref, w1_ref, b1_ref,
                      o_ref):
    x = x_ref[...]
    h = jnp.dot(x, w0_ref[...], preferred_element_type=jnp.float32) + b0_ref[...]
    for _ in range(_NUM_LAYERS):
        h0 = h
        h = jnp.dot(h, ws_ref[...], preferred_element_type=jnp.float32) + bs_ref[...]
        h = jnp.maximum(h, 0.0) + h0
    o_ref[...] = jnp.dot(h, w1_ref[...], preferred_element_type=jnp.float32) + b1_ref[...]


@functools.partial(jax.jit, static_argnames=())
def kernel(batch, feat, W0, b0, Ws, bs, W1, b1):
    del batch
    n, d_in = feat.shape
    d_mid = W0.shape[1]
    d_out = W1.shape[1]
    bn = _BLOCK_ROWS
    assert n % bn == 0

    b0_2d = b0.reshape(1, d_mid)
    bs_2d = bs.reshape(1, d_mid)
    b1_2d = b1.reshape(1, d_out)

    full = lambda shape: pl.BlockSpec(shape, lambda i: (0, 0))
    out = pl.pallas_call(
        _mlp_chain_kernel,
        grid=(n // bn,),
        in_specs=[
            pl.BlockSpec((bn, d_in), lambda i: (i, 0)),
            full((d_in, d_mid)),
            full((1, d_mid)),
            full((d_mid, d_mid)),
            full((1, d_mid)),
            full((d_mid, d_out)),
            full((1, d_out)),
        ],
        out_specs=pl.BlockSpec((bn, d_out), lambda i: (i, 0)),
        out_shape=jax.ShapeDtypeStruct((n, d_out), feat.dtype),
    )(feat, W0, b0_2d, Ws, bs_2d, W1, b1_2d)
    return out
